# Initial kernel scaffold; baseline (speedup 1.0000x reference)
import jax
import jax.numpy as jnp
from jax.experimental import pallas as pl


def kernel(positional_ids_0, positional_ids_1, positional_ids_2, attention_mask, W0, W1, W2):
    raise NotImplementedError("write your pallas kernel here")


# speedup vs baseline: 4.8305x; 4.8305x over previous
import functools

import jax
import jax.numpy as jnp
from jax import lax
from jax.experimental import pallas as pl
from jax.experimental.pallas import tpu as pltpu
from jax.experimental.pallas import tpu_sc as plsc

B, L = 4096, 50
D0, D1, D2 = 64, 32, 32
DOUT = D0 + D1 + D2
N = B * L

_info = plsc.get_sparse_core_info()
NC, NS, LANES = _info.num_cores, _info.num_subcores, _info.num_lanes
NW = NC * NS
PER_W = N // NW
CH = 256
N_CHUNKS = PER_W // CH

_GATHER_DNUMS = lax.GatherDimensionNumbers(
    offset_dims=(), collapsed_slice_dims=(0,), start_index_map=(0,))


def _sc_body(ids0_hbm, ids1_hbm, ids2_hbm, mask_hbm, w0_hbm, w1_hbm, w2_hbm,
             out_hbm, idx0_v, idx1_v, idx2_v, mask_v, e0_v, e1_v, e2_v,
             out_v, sem):
    wid = lax.axis_index("s") * NC + lax.axis_index("c")
    base_w = wid * PER_W

    def chunk_body(ci, carry):
        base = base_w + ci * CH
        pltpu.sync_copy(ids0_hbm.at[pl.ds(base, CH)], idx0_v)
        pltpu.sync_copy(ids1_hbm.at[pl.ds(base, CH)], idx1_v)
        pltpu.sync_copy(ids2_hbm.at[pl.ds(base, CH)], idx2_v)
        pltpu.sync_copy(mask_hbm.at[pl.ds(base, CH)], mask_v)
        cp0 = pltpu.async_copy(w0_hbm.at[idx0_v], e0_v, sem)
        cp1 = pltpu.async_copy(w1_hbm.at[idx1_v], e1_v, sem)
        cp2 = pltpu.async_copy(w2_hbm.at[idx2_v], e2_v, sem)
        cp0.wait()
        cp1.wait()
        cp2.wait()

        def group_body(g, gcarry):
            m16 = mask_v[pl.ds(g * LANES, LANES)]
            for lane in range(LANES):
                r = g * LANES + lane
                m = lax.gather(
                    m16, jnp.full((LANES, 1), lane, dtype=jnp.int32),
                    _GATHER_DNUMS, slice_sizes=(1,),
                    mode=lax.GatherScatterMode.PROMISE_IN_BOUNDS)
                for j in range(D0 // LANES):
                    out_v[r, pl.ds(j * LANES, LANES)] = (
                        e0_v[r, pl.ds(j * LANES, LANES)] * m)
                for j in range(D1 // LANES):
                    out_v[r, pl.ds(D0 + j * LANES, LANES)] = (
                        e1_v[r, pl.ds(j * LANES, LANES)] * m)
                for j in range(D2 // LANES):
                    out_v[r, pl.ds(D0 + D1 + j * LANES, LANES)] = (
                        e2_v[r, pl.ds(j * LANES, LANES)] * m)
            return gcarry

        lax.fori_loop(0, CH // LANES, group_body, 0)
        pltpu.sync_copy(out_v, out_hbm.at[pl.ds(base, CH)])
        return carry

    lax.fori_loop(0, N_CHUNKS, chunk_body, 0)


_sc_call = functools.partial(
    pl.kernel,
    out_type=jax.ShapeDtypeStruct((N, DOUT), jnp.float32),
    mesh=plsc.VectorSubcoreMesh(core_axis_name="c", subcore_axis_name="s"),
    compiler_params=pltpu.CompilerParams(use_tc_tiling_on_sc=False),
    scratch_types=[
        pltpu.VMEM((CH,), jnp.int32),
        pltpu.VMEM((CH,), jnp.int32),
        pltpu.VMEM((CH,), jnp.int32),
        pltpu.VMEM((CH,), jnp.float32),
        pltpu.VMEM((CH, D0), jnp.float32),
        pltpu.VMEM((CH, D1), jnp.float32),
        pltpu.VMEM((CH, D2), jnp.float32),
        pltpu.VMEM((CH, DOUT), jnp.float32),
        pltpu.SemaphoreType.DMA,
    ],
)(_sc_body)


def kernel(positional_ids_0, positional_ids_1, positional_ids_2,
           attention_mask, W0, W1, W2):
    ids0 = positional_ids_0.reshape(N).astype(jnp.int32)
    ids1 = positional_ids_1.reshape(N).astype(jnp.int32)
    ids2 = positional_ids_2.reshape(N).astype(jnp.int32)
    mask = attention_mask.reshape(N).astype(jnp.float32)
    out = _sc_call(ids0, ids1, ids2, mask, W0, W1, W2)
    return out.reshape(B, L, DOUT)


# speedup vs baseline: 6.1028x; 1.2634x over previous
import functools

import jax
import jax.numpy as jnp
from jax import lax
from jax.experimental import pallas as pl
from jax.experimental.pallas import tpu as pltpu
from jax.experimental.pallas import tpu_sc as plsc

B, L = 4096, 50
D0, D1, D2 = 64, 32, 32
DOUT = D0 + D1 + D2
N = B * L

_info = plsc.get_sparse_core_info()
NC, NS, LANES = _info.num_cores, _info.num_subcores, _info.num_lanes
NW = NC * NS
PER_W = N // NW
CH = 400
N_CHUNKS = PER_W // CH
NBUF = 2


def _sc_body(ids0_hbm, ids1_hbm, ids2_hbm, w0_hbm, w1_hbm, w2_hbm,
             out_hbm, idx0_v, idx1_v, idx2_v, e0_v, e1_v, e2_v, gsem, wsem):
    wid = lax.axis_index("s") * NC + lax.axis_index("c")
    base_w = wid * PER_W

    def stage_and_gather(ci, s):
        tok = base_w + ci * CH
        pltpu.sync_copy(ids0_hbm.at[pl.ds(tok, CH)], idx0_v.at[s])
        pltpu.sync_copy(ids1_hbm.at[pl.ds(tok, CH)], idx1_v.at[s])
        pltpu.sync_copy(ids2_hbm.at[pl.ds(tok, CH)], idx2_v.at[s])
        cp0 = pltpu.async_copy(w0_hbm.at[idx0_v.at[s]], e0_v.at[s], gsem)
        cp1 = pltpu.async_copy(w1_hbm.at[idx1_v.at[s]], e1_v.at[s], gsem)
        cp2 = pltpu.async_copy(w2_hbm.at[idx2_v.at[s]], e2_v.at[s], gsem)
        return cp0, cp1, cp2

    def fire_write(ci, s):
        tok = base_w + ci * CH
        rows = pl.ds(tok, CH)
        pltpu.async_copy(e0_v.at[s], out_hbm.at[rows, pl.ds(0, D0)], wsem)
        pltpu.async_copy(e1_v.at[s], out_hbm.at[rows, pl.ds(D0, D1)], wsem)
        pltpu.async_copy(e2_v.at[s],
                         out_hbm.at[rows, pl.ds(D0 + D1, D2)], wsem)

    def drain_write(ci, s):
        tok = base_w + ci * CH
        rows = pl.ds(tok, CH)
        pltpu.make_async_copy(e0_v.at[s],
                              out_hbm.at[rows, pl.ds(0, D0)], wsem).wait()
        pltpu.make_async_copy(e1_v.at[s],
                              out_hbm.at[rows, pl.ds(D0, D1)], wsem).wait()
        pltpu.make_async_copy(e2_v.at[s],
                              out_hbm.at[rows, pl.ds(D0 + D1, D2)],
                              wsem).wait()

    def pair_body(p, carry):
        for s in range(NBUF):
            ci = p * NBUF + s

            @pl.when(p > 0)
            def _():
                drain_write(ci, s)

            cps = stage_and_gather(ci, s)
            for cp in cps:
                cp.wait()
            fire_write(ci, s)
        return carry

    lax.fori_loop(0, N_CHUNKS // NBUF, pair_body, 0)
    for s in range(NBUF):
        drain_write(0, s)


_sc_call = functools.partial(
    pl.kernel,
    out_type=jax.ShapeDtypeStruct((N, DOUT), jnp.float32),
    mesh=plsc.VectorSubcoreMesh(core_axis_name="c", subcore_axis_name="s"),
    compiler_params=pltpu.CompilerParams(use_tc_tiling_on_sc=False),
    scratch_types=[
        pltpu.VMEM((NBUF, CH), jnp.int32),
        pltpu.VMEM((NBUF, CH), jnp.int32),
        pltpu.VMEM((NBUF, CH), jnp.int32),
        pltpu.VMEM((NBUF, CH, D0), jnp.float32),
        pltpu.VMEM((NBUF, CH, D1), jnp.float32),
        pltpu.VMEM((NBUF, CH, D2), jnp.float32),
        pltpu.SemaphoreType.DMA,
        pltpu.SemaphoreType.DMA,
    ],
)(_sc_body)

BB = 16


def _tc_body(g_ref, m_ref, o_ref):
    for bi in range(BB):
        m = m_ref[bi, :]
        o_ref[bi] = g_ref[pl.ds(bi * L, L), :] * m[:, None]


_tc_call = pl.pallas_call(
    _tc_body,
    grid=(B // BB,),
    in_specs=[
        pl.BlockSpec((BB * L, DOUT), lambda i: (i, 0)),
        pl.BlockSpec((BB, L), lambda i: (i, 0)),
    ],
    out_specs=pl.BlockSpec((BB, L, DOUT), lambda i: (i, 0, 0)),
    out_shape=jax.ShapeDtypeStruct((B, L, DOUT), jnp.float32),
)


def kernel(positional_ids_0, positional_ids_1, positional_ids_2,
           attention_mask, W0, W1, W2):
    ids0 = positional_ids_0.reshape(N).astype(jnp.int32)
    ids1 = positional_ids_1.reshape(N).astype(jnp.int32)
    ids2 = positional_ids_2.reshape(N).astype(jnp.int32)
    gathered = _sc_call(ids0, ids1, ids2, W0, W1, W2)
    return _tc_call(gathered, attention_mask)


# speedup vs baseline: 7.8683x; 1.2893x over previous
import functools

import jax
import jax.numpy as jnp
from jax import lax
from jax.experimental import pallas as pl
from jax.experimental.pallas import tpu as pltpu
from jax.experimental.pallas import tpu_sc as plsc

B, L = 4096, 50
LP = 56
D0, D1, D2 = 64, 32, 32
DOUT = D0 + D1 + D2
N = B * L

_info = plsc.get_sparse_core_info()
NC, NS, LANES = _info.num_cores, _info.num_subcores, _info.num_lanes
NW = NC * NS
BATCH_PER_W = B // NW
CB = 8
CH = CB * L
N_CHUNKS = BATCH_PER_W // CB
NBUF = 2


def _sc_body(ids0_hbm, ids1_hbm, ids2_hbm, w0_hbm, w1_hbm, w2_hbm,
             out_hbm, idx0_v, idx1_v, idx2_v, e0_v, e1_v, e2_v, gsem, wsem):
    wid = lax.axis_index("s") * NC + lax.axis_index("c")
    batch_w = wid * BATCH_PER_W

    def stage_and_gather(ci, s):
        tok = (batch_w + ci * CB) * L
        pltpu.sync_copy(ids0_hbm.at[pl.ds(tok, CH)], idx0_v.at[s])
        pltpu.sync_copy(ids1_hbm.at[pl.ds(tok, CH)], idx1_v.at[s])
        pltpu.sync_copy(ids2_hbm.at[pl.ds(tok, CH)], idx2_v.at[s])
        cp0 = pltpu.async_copy(w0_hbm.at[idx0_v.at[s]], e0_v.at[s], gsem)
        cp1 = pltpu.async_copy(w1_hbm.at[idx1_v.at[s]], e1_v.at[s], gsem)
        cp2 = pltpu.async_copy(w2_hbm.at[idx2_v.at[s]], e2_v.at[s], gsem)
        return cp0, cp1, cp2

    def batch_writes(ci, s):
        b0 = batch_w + ci * CB
        for bi in range(CB):
            rows = pl.ds(bi * L, L)
            yield (e0_v.at[s, rows], out_hbm.at[b0 + bi, pl.ds(0, L),
                                                pl.ds(0, D0)])
            yield (e1_v.at[s, rows], out_hbm.at[b0 + bi, pl.ds(0, L),
                                                pl.ds(D0, D1)])
            yield (e2_v.at[s, rows], out_hbm.at[b0 + bi, pl.ds(0, L),
                                                pl.ds(D0 + D1, D2)])

    def fire_write(ci, s):
        for src, dst in batch_writes(ci, s):
            pltpu.async_copy(src, dst, wsem)

    def drain_write(ci, s):
        for src, dst in batch_writes(ci, s):
            pltpu.make_async_copy(src, dst, wsem).wait()

    def pair_body(p, carry):
        for s in range(NBUF):
            ci = p * NBUF + s

            @pl.when(p > 0)
            def _():
                drain_write(ci, s)

            cps = stage_and_gather(ci, s)
            for cp in cps:
                cp.wait()
            fire_write(ci, s)
        return carry

    lax.fori_loop(0, N_CHUNKS // NBUF, pair_body, 0)
    for s in range(NBUF):
        drain_write(0, s)


_sc_call = functools.partial(
    pl.kernel,
    out_type=jax.ShapeDtypeStruct((B, LP, DOUT), jnp.float32),
    mesh=plsc.VectorSubcoreMesh(core_axis_name="c", subcore_axis_name="s"),
    compiler_params=pltpu.CompilerParams(use_tc_tiling_on_sc=False),
    scratch_types=[
        pltpu.VMEM((NBUF, CH), jnp.int32),
        pltpu.VMEM((NBUF, CH), jnp.int32),
        pltpu.VMEM((NBUF, CH), jnp.int32),
        pltpu.VMEM((NBUF, CH, D0), jnp.float32),
        pltpu.VMEM((NBUF, CH, D1), jnp.float32),
        pltpu.VMEM((NBUF, CH, D2), jnp.float32),
        pltpu.SemaphoreType.DMA,
        pltpu.SemaphoreType.DMA,
    ],
)(_sc_body)

BB = 128


def _tc_body(g_ref, mt_ref, o_ref):
    for bi in range(BB):
        m = mt_ref[:, bi]
        o_ref[bi] = g_ref[bi, pl.ds(0, L), :] * m[:, None]


_tc_call = pl.pallas_call(
    _tc_body,
    grid=(B // BB,),
    in_specs=[
        pl.BlockSpec((BB, LP, DOUT), lambda i: (i, 0, 0)),
        pl.BlockSpec((L, BB), lambda i: (0, i)),
    ],
    out_specs=pl.BlockSpec((BB, L, DOUT), lambda i: (i, 0, 0)),
    out_shape=jax.ShapeDtypeStruct((B, L, DOUT), jnp.float32),
)


def kernel(positional_ids_0, positional_ids_1, positional_ids_2,
           attention_mask, W0, W1, W2):
    ids0 = positional_ids_0.reshape(N).astype(jnp.int32)
    ids1 = positional_ids_1.reshape(N).astype(jnp.int32)
    ids2 = positional_ids_2.reshape(N).astype(jnp.int32)
    gathered = _sc_call(ids0, ids1, ids2, W0, W1, W2)
    return _tc_call(gathered, attention_mask.T)
